# baseline (speedup 1.0000x reference)
import functools

import jax
import jax.numpy as jnp
from jax import lax
from jax.experimental import pallas as pl
from jax.experimental.pallas import tpu as pltpu
from jax.experimental.pallas import tpu_sc as plsc

EMB = 64
LANES = 16
NUM_WORKERS = 32
BSLICE = 128
NBUF = 7
LOOK_L = 6
LOOK_G = 5


ROWPITCH = 72


def _body(inp_hbm, lab_hbm, tab_hbm, out_hbm, idx_v, rows_v, outb_v,
          rows_p, si, sg, so, *, hist):
    wid = lax.axis_index("s") * 2 + lax.axis_index("c")

    def issue_loads(h, q):
        hs = h // 8
        hr = h % 8 if isinstance(h, int) else lax.rem(h, 8)
        pltpu.async_copy(lab_hbm.at[hs, wid, hr], idx_v[q], si)
        pltpu.async_copy(inp_hbm.at[h, :, wid], outb_v[q], si)

    def wait_loads(h, q):
        hs = h // 8
        hr = h % 8 if isinstance(h, int) else lax.rem(h, 8)
        pltpu.make_async_copy(lab_hbm.at[hs, wid, hr], idx_v[q], si).wait()
        pltpu.make_async_copy(inp_hbm.at[h, :, wid], outb_v[q], si).wait()

    def issue_gather(q):
        pltpu.async_copy(tab_hbm.at[idx_v[q]], rows_v[q], sg)

    def wait_gather(q):
        pltpu.make_async_copy(tab_hbm.at[idx_v[q]], rows_v[q], sg).wait()

    def issue_out(h, q):
        pltpu.async_copy(outb_v[q], out_hbm.at[h, :, wid], so)

    def wait_out(h, q):
        pltpu.make_async_copy(outb_v[q], out_hbm.at[h, :, wid], so).wait()

    bidx = [lax.iota(jnp.int32, LANES) + g * LANES
            for g in range(BSLICE // LANES)]

    def compute(q):
        rows_q = rows_v[q]
        outb_q = outb_v[q]

        @plsc.parallel_loop(0, BSLICE, unroll=4)
        def _(b):
            for g in range(EMB // LANES):
                sl = pl.ds(g * LANES, LANES)
                rows_p[b, sl] = rows_q[b, sl]

        @plsc.parallel_loop(0, EMB, unroll=2)
        def _(d):
            s = lax.shift_right_logical(d, 3)
            r = lax.bitwise_and(d, 7)
            dcol = jnp.zeros((LANES,), jnp.int32) + d
            for g in range(BSLICE // LANES):
                emb = plsc.load_gather(rows_p, [bidx[g], dcol])
                plsc.addupdate(outb_q.at[s, r, pl.ds(g * LANES, LANES)], emb)

    def _when(cond, fn):
        if isinstance(cond, bool):
            if cond:
                fn()
        else:
            pl.when(cond)(fn)

    def step(h, q):
        def _feed():
            wait_loads(h + LOOK_G, (q + LOOK_G) % NBUF)
            issue_gather((q + LOOK_G) % NBUF)

        _when(h + LOOK_G < hist, _feed)
        wait_gather(q)
        compute(q)
        issue_out(h, q)
        _when(h >= 1, lambda: wait_out(h - 1, (q - 1) % NBUF))
        _when(h + LOOK_L < hist,
              lambda: issue_loads(h + LOOK_L, (q + LOOK_L) % NBUF))

    for k in range(LOOK_L):
        issue_loads(k, k)
    for k in range(LOOK_G):
        wait_loads(k, k)
        issue_gather(k)

    def multi_step(j, carry):
        h = j * NBUF
        for q in range(NBUF):
            step(h + q, q)
        return carry

    main_steps = (hist // NBUF) * NBUF
    lax.fori_loop(0, hist // NBUF, multi_step, 0)
    for h in range(main_steps, hist):
        step(h, h % NBUF)
    wait_out(hist - 1, (hist - 1) % NBUF)


TP_BLK = 128
TP_PITCH = 136
TP_NBUF = 3


def _tbody(tab_hbm, out_hbm, slabs, outs, slab_p, slab_t, out_t, si, so,
           *, vocab):
    wid = lax.axis_index("s") * 2 + lax.axis_index("c")
    num_blocks = vocab // TP_BLK
    jmax = (num_blocks + NUM_WORKERS - 1) // NUM_WORKERS

    def blk(j):
        return wid + NUM_WORKERS * j

    def issue_in(j, q):
        b = blk(j)
        pltpu.async_copy(tab_hbm.at[:, pl.ds(b * TP_BLK, TP_BLK)],
                         slabs[q], si)

    def wait_in(j, q):
        b = blk(j)
        pltpu.make_async_copy(tab_hbm.at[:, pl.ds(b * TP_BLK, TP_BLK)],
                              slabs[q], si).wait()

    def issue_out(j, q):
        b = blk(j)
        pltpu.async_copy(outs[q],
                         out_hbm.at[pl.ds(b * (TP_BLK // 2), TP_BLK // 2)],
                         so)

    def wait_out(j, q):
        b = blk(j)
        pltpu.make_async_copy(
            outs[q], out_hbm.at[pl.ds(b * (TP_BLK // 2), TP_BLK // 2)],
            so).wait()

    dvecs = [lax.iota(jnp.int32, LANES) + g * LANES
             for g in range(EMB // LANES)]

    def transpose_into(dst, npairs):
        @plsc.parallel_loop(0, npairs, unroll=2)
        def _(j):
            for h in range(2):
                cvec = jnp.zeros((LANES,), jnp.int32) + (2 * j + h)
                for g in range(EMB // LANES):
                    v = plsc.load_gather(slab_p, [dvecs[g], cvec])
                    dst[j, pl.ds(h * EMB + g * LANES, LANES)] = v

    def compute(q):
        @plsc.parallel_loop(0, EMB, unroll=4)
        def _(d):
            for g in range(TP_BLK // LANES):
                sl = pl.ds(g * LANES, LANES)
                slab_p[d, sl] = slabs[q][d, sl]

        transpose_into(outs[q], TP_BLK // 2)

    def _when(cond, fn):
        if isinstance(cond, bool):
            if cond:
                fn()
        else:
            pl.when(cond)(fn)

    def valid(j):
        return blk(j) < num_blocks

    def tstep(j, q):
        def _work():
            wait_in(j, q)
            compute(q)
            issue_out(j, q)

        pl.when(valid(j))(_work)
        _when(j >= 1,
              lambda: pl.when(valid(j - 1))(
                  lambda: wait_out(j - 1, (q - 1) % TP_NBUF)))
        _when(j + TP_NBUF - 1 < jmax,
              lambda: pl.when(valid(j + TP_NBUF - 1))(
                  lambda: issue_in(j + TP_NBUF - 1,
                                   (q + TP_NBUF - 1) % TP_NBUF)))

    for k in range(TP_NBUF - 1):
        pl.when(valid(k))(lambda k=k: issue_in(k, k))

    def tmulti(i, carry):
        j = i * TP_NBUF
        for q in range(TP_NBUF):
            tstep(j + q, q)
        return carry

    main_j = (jmax // TP_NBUF) * TP_NBUF
    lax.fori_loop(0, jmax // TP_NBUF, tmulti, 0)
    for j in range(main_j, jmax):
        tstep(j, j % TP_NBUF)
    pl.when(valid(jmax - 1))(
        lambda: wait_out(jmax - 1, (jmax - 1) % TP_NBUF))

    tail = vocab % TP_BLK
    if tail:
        def _tail():
            pltpu.sync_copy(tab_hbm.at[:, pl.ds(vocab - tail, tail)], slab_t)

            @plsc.parallel_loop(0, EMB, unroll=4)
            def _(d):
                for g in range(tail // LANES):
                    sl = pl.ds(g * LANES, LANES)
                    slab_p[d, sl] = slab_t[d, sl]

            transpose_into(out_t, tail // 2)
            pltpu.sync_copy(
                out_t, out_hbm.at[pl.ds((vocab - tail) // 2, tail // 2)])

        pl.when(wid == 0)(_tail)


def _transpose_table(emb_table):
    vocab, emb = emb_table.shape
    tail = vocab % TP_BLK
    assert emb == EMB and tail % LANES == 0
    tab_t = jnp.transpose(emb_table, (1, 0))
    mesh = plsc.VectorSubcoreMesh(core_axis_name="c", subcore_axis_name="s")
    run = pl.kernel(
        functools.partial(_tbody, vocab=vocab),
        out_type=jax.ShapeDtypeStruct((vocab // 2, 2 * EMB), jnp.float32),
        mesh=mesh,
        scratch_types=(
            [[pltpu.VMEM((EMB, TP_BLK), jnp.float32)
              for _ in range(TP_NBUF)],
             [pltpu.VMEM((TP_BLK // 2, 2 * EMB), jnp.float32)
              for _ in range(TP_NBUF)],
             pltpu.VMEM((EMB, TP_PITCH), jnp.float32),
             pltpu.VMEM((EMB, TP_BLK // 2), jnp.float32),
             pltpu.VMEM((TP_BLK // 4, 2 * EMB), jnp.float32)]
            + [pltpu.SemaphoreType.DMA] * 2
        ),
        compiler_params=pltpu.CompilerParams(use_tc_tiling_on_sc=True,
                                             needs_layout_passes=False),
    )
    return run(tab_t)


def kernel(inputs, labels, emb_table):
    batch, hist, emb = inputs.shape
    assert emb == EMB and batch == NUM_WORKERS * BSLICE
    VOCAB_ROWS = emb_table.shape[0]

    inp5 = jnp.transpose(inputs, (1, 2, 0))
    inp5 = inp5.reshape(hist, 8, EMB // 8, NUM_WORKERS, BSLICE)
    inp5 = jnp.transpose(inp5, (0, 1, 3, 2, 4))

    lab4 = jnp.transpose(labels, (1, 0)).astype(jnp.int32)
    lab4 = lab4.reshape(hist // 8, 8, NUM_WORKERS, BSLICE)
    lab4 = jnp.transpose(lab4, (0, 2, 1, 3))

    mesh = plsc.VectorSubcoreMesh(core_axis_name="c", subcore_axis_name="s")
    run = pl.kernel(
        functools.partial(_body, hist=hist),
        out_type=jax.ShapeDtypeStruct((hist, 8, NUM_WORKERS, EMB // 8, BSLICE),
                                      jnp.float32),
        mesh=mesh,
        scratch_types=(
            [[pltpu.VMEM((BSLICE,), jnp.int32) for _ in range(NBUF)],
             [pltpu.VMEM((BSLICE, EMB), jnp.float32) for _ in range(NBUF)],
             [pltpu.VMEM((EMB // 8, 8, BSLICE), jnp.float32)
              for _ in range(NBUF)],
             pltpu.VMEM((BSLICE, ROWPITCH), jnp.float32)]
            + [pltpu.SemaphoreType.DMA] * 3
        ),
        compiler_params=pltpu.CompilerParams(use_tc_tiling_on_sc=False,
                                             needs_layout_passes=False),
    )
    tab_rm = _transpose_table(emb_table).reshape(VOCAB_ROWS, EMB)
    out5 = run(inp5, lab4, tab_rm)
    out = jnp.transpose(out5, (0, 1, 3, 2, 4)).reshape(hist, EMB, batch)
    return jnp.transpose(out, (2, 0, 1))


# speedup vs baseline: 1.3528x; 1.3528x over previous
import functools

import jax
import jax.numpy as jnp
from jax import lax
from jax.experimental import pallas as pl
from jax.experimental.pallas import tpu as pltpu
from jax.experimental.pallas import tpu_sc as plsc

EMB = 64
LANES = 16
NUM_WORKERS = 32
BSLICE = 128
NBUF = 7
LOOK_L = 6
LOOK_G = 5


ROWPITCH = 72


def _body(inp_hbm, lab_hbm, tab_hbm, out_hbm, idx_v, rows_v, outb_v,
          rows_p, si, sg, so, *, hist):
    wid = lax.axis_index("s") * 2 + lax.axis_index("c")

    def issue_loads(h, q):
        hs = h // 8
        hr = h % 8 if isinstance(h, int) else lax.rem(h, 8)
        pltpu.async_copy(lab_hbm.at[hs, wid, hr], idx_v[q], si)
        pltpu.async_copy(inp_hbm.at[h, :, wid], outb_v[q], si)

    def wait_loads(h, q):
        hs = h // 8
        hr = h % 8 if isinstance(h, int) else lax.rem(h, 8)
        pltpu.make_async_copy(lab_hbm.at[hs, wid, hr], idx_v[q], si).wait()
        pltpu.make_async_copy(inp_hbm.at[h, :, wid], outb_v[q], si).wait()

    def double_idx(q):
        for g in range(BSLICE // LANES):
            sl = pl.ds(g * LANES, LANES)
            idx_v[q][sl] = lax.shift_left(idx_v[q][sl], 1)

    def issue_gather(q):
        pltpu.async_copy(tab_hbm.at[idx_v[q]], rows_v[q], sg)

    def wait_gather(q):
        pltpu.make_async_copy(tab_hbm.at[idx_v[q]], rows_v[q], sg).wait()

    def issue_out(h, q):
        pltpu.async_copy(outb_v[q], out_hbm.at[h, :, wid], so)

    def wait_out(h, q):
        pltpu.make_async_copy(outb_v[q], out_hbm.at[h, :, wid], so).wait()

    bidx = [lax.iota(jnp.int32, LANES) + g * LANES
            for g in range(BSLICE // LANES)]

    def compute(q):
        rows_q = rows_v[q]
        outb_q = outb_v[q]

        @plsc.parallel_loop(0, BSLICE, unroll=8)
        def _(b):
            for g in range(EMB // LANES):
                sl = pl.ds(g * LANES, LANES)
                rows_p[b, sl] = rows_q[b, sl]

        @plsc.parallel_loop(0, EMB, unroll=4)
        def _(d):
            s = lax.shift_right_logical(d, 3)
            r = lax.bitwise_and(d, 7)
            dcol = jnp.zeros((LANES,), jnp.int32) + d
            for g in range(BSLICE // LANES):
                emb = plsc.load_gather(rows_p, [bidx[g], dcol])
                plsc.addupdate(outb_q.at[s, r, pl.ds(g * LANES, LANES)], emb)

    def _when(cond, fn):
        if isinstance(cond, bool):
            if cond:
                fn()
        else:
            pl.when(cond)(fn)

    def step(h, q):
        def _feed():
            wait_loads(h + LOOK_G, (q + LOOK_G) % NBUF)
            double_idx((q + LOOK_G) % NBUF)
            issue_gather((q + LOOK_G) % NBUF)

        _when(h + LOOK_G < hist, _feed)
        wait_gather(q)
        compute(q)
        issue_out(h, q)
        _when(h >= 1, lambda: wait_out(h - 1, (q - 1) % NBUF))
        _when(h + LOOK_L < hist,
              lambda: issue_loads(h + LOOK_L, (q + LOOK_L) % NBUF))

    for k in range(LOOK_L):
        issue_loads(k, k)
    for k in range(LOOK_G):
        wait_loads(k, k)
        double_idx(k)
        issue_gather(k)

    def multi_step(j, carry):
        h = j * NBUF
        for q in range(NBUF):
            step(h + q, q)
        return carry

    main_steps = (hist // NBUF) * NBUF
    lax.fori_loop(0, hist // NBUF, multi_step, 0)
    for h in range(main_steps, hist):
        step(h, h % NBUF)
    wait_out(hist - 1, (hist - 1) % NBUF)


def kernel(inputs, labels, emb_table):
    batch, hist, emb = inputs.shape
    assert emb == EMB and batch == NUM_WORKERS * BSLICE
    VOCAB_ROWS = emb_table.shape[0]

    inp5 = jnp.transpose(inputs, (1, 2, 0))
    inp5 = inp5.reshape(hist, 8, EMB // 8, NUM_WORKERS, BSLICE)
    inp5 = jnp.transpose(inp5, (0, 1, 3, 2, 4))

    lab4 = jnp.transpose(labels, (1, 0)).astype(jnp.int32)
    lab4 = lab4.reshape(hist // 8, 8, NUM_WORKERS, BSLICE)
    lab4 = jnp.transpose(lab4, (0, 2, 1, 3))

    mesh = plsc.VectorSubcoreMesh(core_axis_name="c", subcore_axis_name="s")
    run = pl.kernel(
        functools.partial(_body, hist=hist),
        out_type=jax.ShapeDtypeStruct((hist, 8, NUM_WORKERS, EMB // 8, BSLICE),
                                      jnp.float32),
        mesh=mesh,
        scratch_types=(
            [[pltpu.VMEM((BSLICE,), jnp.int32) for _ in range(NBUF)],
             [pltpu.VMEM((BSLICE, EMB), jnp.float32) for _ in range(NBUF)],
             [pltpu.VMEM((EMB // 8, 8, BSLICE), jnp.float32)
              for _ in range(NBUF)],
             pltpu.VMEM((BSLICE, ROWPITCH), jnp.float32)]
            + [pltpu.SemaphoreType.DMA] * 3
        ),
        compiler_params=pltpu.CompilerParams(use_tc_tiling_on_sc=False,
                                             needs_layout_passes=False),
    )
    tab_wide = jnp.pad(emb_table, ((0, 0), (0, EMB)))
    tab2 = tab_wide.reshape(2 * VOCAB_ROWS, EMB)
    out5 = run(inp5, lab4, tab2)
    out = jnp.transpose(out5, (0, 1, 3, 2, 4)).reshape(hist, EMB, batch)
    return jnp.transpose(out, (2, 0, 1))
